# baseline (speedup 1.0000x reference)
import functools

import jax
import jax.numpy as jnp
from jax import lax
from jax.experimental import pallas as pl
from jax.experimental.pallas import tpu as pltpu
from jax.experimental.pallas import tpu_sc as plsc

_NC = 2
_NS = 16
_NW = _NC * _NS

_G = 4


@functools.cache
def _build(batch: int, hist: int, vocab: int, dim: int):
    rows_per_w = batch // _NW
    iters = rows_per_w // _G

    mesh = plsc.VectorSubcoreMesh(core_axis_name="c", subcore_axis_name="s")

    @functools.partial(
        pl.kernel,
        mesh=mesh,
        compiler_params=pltpu.CompilerParams(use_tc_tiling_on_sc=False),
        out_type=jax.ShapeDtypeStruct((batch, hist, dim), jnp.float32),
        scratch_types=[
            pltpu.VMEM((2, _G, hist), jnp.int32),
            pltpu.VMEM((2, _G, hist, dim), jnp.float32),
            pltpu.SemaphoreType.DMA,
            pltpu.SemaphoreType.DMA,
        ],
    )
    def gather_kernel(idx_hbm, table_hbm, out_hbm, idx_v, rows_v, sem0, sem1):
        wid = lax.axis_index("s") * _NC + lax.axis_index("c")
        row0 = wid * rows_per_w
        sems = (sem0, sem1)

        def load_fire(t, s):
            row = row0 + t * _G
            pltpu.sync_copy(idx_hbm.at[pl.ds(row, _G)], idx_v.at[s])
            for g in range(_G):
                pltpu.async_copy(
                    table_hbm.at[idx_v.at[s, g]],
                    rows_v.at[s, g],
                    sems[s],
                )

        def drain(s):
            for g in range(_G):
                pltpu.make_async_copy(
                    table_hbm.at[idx_v.at[s, g]],
                    rows_v.at[s, g],
                    sems[s],
                ).wait()

        def write_out(t, s):
            row = row0 + t * _G
            pltpu.sync_copy(rows_v.at[s], out_hbm.at[pl.ds(row, _G)])

        load_fire(0, 0)

        def body(i, carry):
            t0 = 2 * i
            load_fire(t0 + 1, 1)
            drain(0)
            write_out(t0, 0)

            @pl.when(i < iters // 2 - 1)
            def _():
                load_fire(t0 + 2, 0)

            drain(1)
            write_out(t0 + 1, 1)
            return carry

        lax.fori_loop(0, iters // 2, body, 0)

    return gather_kernel


def kernel(input_, weight):
    batch, hist = input_.shape
    vocab, dim = weight.shape
    idx = input_.astype(jnp.int32)
    return _build(batch, hist, vocab, dim)(idx, weight)


# speedup vs baseline: 1.6481x; 1.6481x over previous
import functools

import jax
import jax.numpy as jnp
from jax import lax
from jax.experimental import pallas as pl
from jax.experimental.pallas import tpu as pltpu
from jax.experimental.pallas import tpu_sc as plsc

_NC = 2
_NS = 16
_NW = _NC * _NS

_G = 4


@functools.cache
def _build(batch: int, hist: int, vocab: int, dim: int):
    rows_per_w = batch // _NW
    iters = rows_per_w // _G
    n = batch * hist
    gh = _G * hist

    mesh = plsc.VectorSubcoreMesh(core_axis_name="c", subcore_axis_name="s")

    @functools.partial(
        pl.kernel,
        mesh=mesh,
        compiler_params=pltpu.CompilerParams(use_tc_tiling_on_sc=False),
        out_type=jax.ShapeDtypeStruct((n, 2 * dim), jnp.float32),
        scratch_types=[
            pltpu.VMEM((2, _G, hist), jnp.int32),
            pltpu.VMEM((2, gh, dim), jnp.float32),
            pltpu.SemaphoreType.DMA,
            pltpu.SemaphoreType.DMA,
        ],
    )
    def gather_kernel(idx_hbm, table_hbm, out_hbm, idx_v, rows_v, sem0, sem1):
        wid = lax.axis_index("s") * _NC + lax.axis_index("c")
        brow0 = wid * rows_per_w
        sems = (sem0, sem1)

        def load_fire(t, s):
            brow = brow0 + t * _G
            pltpu.sync_copy(idx_hbm.at[pl.ds(brow, _G)], idx_v.at[s])
            for g in range(_G):
                pltpu.async_copy(
                    table_hbm.at[idx_v.at[s, g]],
                    rows_v.at[s, pl.ds(g * hist, hist)],
                    sems[s],
                )

        def drain(s):
            for g in range(_G):
                pltpu.make_async_copy(
                    table_hbm.at[idx_v.at[s, g]],
                    rows_v.at[s, pl.ds(g * hist, hist)],
                    sems[s],
                ).wait()

        def write_out(t, s):
            nrow = (brow0 + t * _G) * hist
            pltpu.sync_copy(
                rows_v.at[s],
                out_hbm.at[pl.ds(nrow, gh), pl.ds(0, dim)],
            )

        load_fire(0, 0)

        def body(i, carry):
            t0 = 2 * i
            load_fire(t0 + 1, 1)
            drain(0)
            write_out(t0, 0)

            @pl.when(i < iters // 2 - 1)
            def _():
                load_fire(t0 + 2, 0)

            drain(1)
            write_out(t0 + 1, 1)
            return carry

        lax.fori_loop(0, iters // 2, body, 0)

    return gather_kernel


def kernel(input_, weight):
    batch, hist = input_.shape
    vocab, dim = weight.shape
    idx = input_.astype(jnp.int32)
    out = _build(batch, hist, vocab, dim)(idx, weight)
    return out[:, :dim].reshape(batch, hist, dim)
